# baseline (speedup 1.0000x reference)
import jax
import jax.numpy as jnp
from jax import lax
from jax.experimental import pallas as pl
from jax.experimental.pallas import tpu as pltpu
from jax.experimental.pallas import tpu_sc as plsc

VOCAB = 1000000
EMBD = 64
B = 4096
L = 50

NW = 32
BPW = B // NW
LANES = 16
NGRP = BPW // LANES


def _emb_body(xt_hbm, tab_hbm, out_hbm, idx_v, gidx_v, pcol_v, rows_v,
              outc_v, gsem, wsem):
    nc = 2
    wid = lax.axis_index("s") * nc + lax.axis_index("c")
    b0 = wid * BPW

    pltpu.sync_copy(xt_hbm.at[:, pl.ds(b0, BPW)], idx_v)

    iotas = [lax.iota(jnp.int32, LANES) + (g * LANES) for g in range(NGRP)]

    def prep(l, p):
        for g in range(NGRP):
            v = idx_v[l, pl.ds(g * LANES, LANES)]
            gidx_v[p, pl.ds(g * LANES, LANES)] = lax.shift_right_logical(v, 1)
            pcol_v[p, pl.ds(g * LANES, LANES)] = lax.shift_left(
                lax.bitwise_and(v, 1), 6)

    def gather(p):
        pltpu.async_copy(tab_hbm.at[gidx_v.at[p]], rows_v.at[p], gsem.at[p])

    def gather_wait(p):
        pltpu.make_async_copy(tab_hbm.at[gidx_v.at[p]], rows_v.at[p],
                              gsem.at[p]).wait()

    def transpose(p):
        for g in range(NGRP):
            rowi = iotas[g]
            cols = pcol_v[p, pl.ds(g * LANES, LANES)]
            for e in range(EMBD):
                val = plsc.load_gather(rows_v.at[p], [rowi, cols + e])
                outc_v[p, e, pl.ds(g * LANES, LANES)] = val

    def write(l, p):
        pltpu.async_copy(outc_v.at[p], out_hbm.at[l, :, pl.ds(b0, BPW)],
                         wsem.at[p])

    def write_wait(l, p):
        pltpu.make_async_copy(outc_v.at[p], out_hbm.at[l, :, pl.ds(b0, BPW)],
                              wsem.at[p]).wait()

    prep(0, 0)
    gather(0)

    @pl.loop(0, L // 2)
    def _(ll):
        for p in range(2):
            l = 2 * ll + p

            gather_wait(p)

            def prefetch(l=l, p=p):
                prep(l + 1, 1 - p)
                gather(1 - p)

            if p == 0:
                prefetch()
            else:
                pl.when(ll < L // 2 - 1)(prefetch)

            def drain(l=l, p=p):
                write_wait(l - 2, p)

            if p == 0:
                pl.when(l >= 2)(drain)
            else:
                pl.when(l >= 2)(drain)

            transpose(p)
            write(l, p)

    write_wait(L - 2, 0)
    write_wait(L - 1, 1)


@jax.jit
def _emb(xt, tab2):
    mesh = plsc.VectorSubcoreMesh(core_axis_name="c", subcore_axis_name="s")
    f = pl.kernel(
        _emb_body,
        out_type=jax.ShapeDtypeStruct((L, EMBD, B), jnp.float32),
        mesh=mesh,
        compiler_params=pltpu.CompilerParams(use_tc_tiling_on_sc=True,
                                             needs_layout_passes=False),
        scratch_types=[
            pltpu.VMEM((L, BPW), jnp.int32),
            pltpu.VMEM((2, BPW), jnp.int32),
            pltpu.VMEM((2, BPW), jnp.int32),
            pltpu.VMEM((2, BPW, 128), jnp.float32),
            pltpu.VMEM((2, EMBD, BPW), jnp.float32),
            pltpu.SemaphoreType.DMA((2,)),
            pltpu.SemaphoreType.DMA((2,)),
        ],
    )
    return f(xt, tab2)


def kernel(x, table):
    xt = jnp.transpose(x.astype(jnp.int32))
    tab2 = jnp.reshape(table, (VOCAB // 2, 2 * EMBD))
    outt = _emb(xt, tab2)
    return jnp.transpose(outt, (2, 0, 1))


# speedup vs baseline: 1.3785x; 1.3785x over previous
import jax
import jax.numpy as jnp
from jax import lax
from jax.experimental import pallas as pl
from jax.experimental.pallas import tpu as pltpu
from jax.experimental.pallas import tpu_sc as plsc

VOCAB = 1000000
EMBD = 64
B = 4096
L = 50

NW = 32
BPW = B // NW


def _emb_body(xt_hbm, tab_hbm, out_hbm, idx_v, rows_v, gsem, wsem):
    nc = 2
    wid = lax.axis_index("s") * nc + lax.axis_index("c")
    b0 = wid * BPW

    pltpu.sync_copy(xt_hbm.at[:, pl.ds(b0, BPW)], idx_v)

    def gather(l, p):
        pltpu.async_copy(tab_hbm.at[idx_v.at[l]], rows_v.at[p], gsem.at[p])

    def gather_wait(l, p):
        pltpu.make_async_copy(tab_hbm.at[idx_v.at[l]], rows_v.at[p],
                              gsem.at[p]).wait()

    def write(l, p):
        pltpu.async_copy(rows_v.at[p], out_hbm.at[pl.ds(b0, BPW), l, :],
                         wsem.at[p])

    def write_wait(l, p):
        pltpu.make_async_copy(rows_v.at[p], out_hbm.at[pl.ds(b0, BPW), l, :],
                              wsem.at[p]).wait()

    gather(0, 0)

    @pl.loop(0, L // 2)
    def _(ll):
        for p in range(2):
            l = 2 * ll + p

            gather_wait(l, p)

            def prefetch(l=l, p=p):
                def drain(l=l, p=p):
                    write_wait(l - 1, 1 - p)

                pl.when(l >= 1)(drain)
                gather(l + 1, 1 - p)

            if p == 0:
                prefetch()
            else:
                pl.when(ll < L // 2 - 1)(prefetch)

            write(l, p)

    write_wait(L - 2, 0)
    write_wait(L - 1, 1)


@jax.jit
def _emb(xt, tab_pad):
    mesh = plsc.VectorSubcoreMesh(core_axis_name="c", subcore_axis_name="s")
    f = pl.kernel(
        _emb_body,
        out_type=jax.ShapeDtypeStruct((B, L, 128), jnp.float32),
        mesh=mesh,
        compiler_params=pltpu.CompilerParams(use_tc_tiling_on_sc=True,
                                             needs_layout_passes=False),
        scratch_types=[
            pltpu.VMEM((L, BPW), jnp.int32),
            pltpu.VMEM((2, BPW, 128), jnp.float32),
            pltpu.SemaphoreType.DMA((2,)),
            pltpu.SemaphoreType.DMA((2,)),
        ],
    )
    return f(xt, tab_pad)


def kernel(x, table):
    xt = jnp.transpose(x.astype(jnp.int32))
    tab_pad = jnp.pad(table, ((0, 0), (0, 128 - EMBD)))
    return _emb(xt, tab_pad)[:, :, 0:EMBD]
